# Initial kernel scaffold; baseline (speedup 1.0000x reference)
import jax
import jax.numpy as jnp
from jax.experimental import pallas as pl


def kernel(vecs, loss_mask):
    raise NotImplementedError("write your pallas kernel here")


# speedup vs baseline: 3.0674x; 3.0674x over previous
import jax
import jax.numpy as jnp
from jax.experimental import pallas as pl
from jax.experimental.pallas import tpu as pltpu

B, H, L, D = 1, 16, 2048, 128
S = 1024
PE_LAM = 10000.0
EPS = 1e-6


def _codebook():
    pos = jnp.arange(S, dtype=jnp.float32)
    i = jnp.arange(D // 2, dtype=jnp.float32)
    denom = jnp.power(jnp.float32(PE_LAM), 2.0 * i / float(D))
    ang = pos[:, None] / denom[None, :]
    embs = jnp.concatenate([jnp.sin(ang), jnp.cos(ang)], axis=-1)
    ms = jnp.mean(jnp.square(embs), axis=-1, keepdims=True)
    c = embs * jax.lax.rsqrt(ms + EPS)
    return (float(D) ** -0.25) * c


def _vq_body(v_ref, cb_ref, cn_ref, mask_ref, hat_ref, z_ref, err_ref, lc_ref):
    h = pl.program_id(0)
    v = v_ref[0]
    cb = cb_ref[...]

    scores = jax.lax.dot_general(
        v, cb, (((1,), (1,)), ((), ())),
        preferred_element_type=jnp.float32,
        precision=jax.lax.Precision.DEFAULT)
    vnorm2 = jnp.sum(v * v, axis=1)
    d = vnorm2[:, None] - 2.0 * scores + cn_ref[...]
    dmin = jnp.min(d, axis=1, keepdims=True)
    iota = jax.lax.broadcasted_iota(jnp.int32, (L, S), 1)
    z = jnp.min(jnp.where(d == dmin, iota, S), axis=1).astype(jnp.int32)

    errs2 = jnp.maximum(dmin[:, 0], 0.0)

    oh = (iota == z[:, None]).astype(jnp.float32)
    hat = jax.lax.dot_general(
        oh, cb, (((1,), (0,)), ((), ())),
        preferred_element_type=jnp.float32,
        precision=jax.lax.Precision.HIGHEST)

    hat_ref[0] = hat
    z_ref[0, 0] = z
    err_ref[0, 0] = errs2

    part = jnp.sum(mask_ref[...] * errs2[None, :]) * (1.0 / float(B * L))

    @pl.when(h == 0)
    def _():
        lc_ref[0, 0] = 0.0

    lc_ref[0, 0] += part


def kernel(vecs, loss_mask):
    cb = _codebook()
    cn2 = jnp.sum(cb * cb, axis=1)[None, :]
    v = vecs.reshape(H, L, D)
    mask = loss_mask.reshape(1, L)

    hat, z3, err3, lc = pl.pallas_call(
        _vq_body,
        grid=(H,),
        in_specs=[
            pl.BlockSpec((1, L, D), lambda h: (h, 0, 0)),
            pl.BlockSpec((S, D), lambda h: (0, 0)),
            pl.BlockSpec((1, S), lambda h: (0, 0)),
            pl.BlockSpec((1, L), lambda h: (0, 0)),
        ],
        out_specs=[
            pl.BlockSpec((1, L, D), lambda h: (h, 0, 0)),
            pl.BlockSpec((1, 1, L), lambda h: (h, 0, 0)),
            pl.BlockSpec((1, 1, L), lambda h: (h, 0, 0)),
            pl.BlockSpec(memory_space=pltpu.SMEM),
        ],
        out_shape=[
            jax.ShapeDtypeStruct((H, L, D), jnp.float32),
            jax.ShapeDtypeStruct((H, 1, L), jnp.int32),
            jax.ShapeDtypeStruct((H, 1, L), jnp.float32),
            jax.ShapeDtypeStruct((1, 1), jnp.float32),
        ],
    )(v, cb, cn2, mask)

    vecs_hat = hat.reshape(B, H, L, D)
    z = z3.reshape(B, H, L)
    errs2 = err3.reshape(B, H, L)
    l_commit = lc[0, 0]
    l_codebook = jnp.zeros((), dtype=jnp.float32)
    return vecs_hat, z, l_commit, l_codebook, errs2


# speedup vs baseline: 5.6441x; 1.8400x over previous
import jax
import jax.numpy as jnp
from jax.experimental import pallas as pl
from jax.experimental.pallas import tpu as pltpu

B, H, L, D = 1, 16, 2048, 128
S = 1024
PE_LAM = 10000.0
EPS = 1e-6


def _codebook():
    pos = jnp.arange(S, dtype=jnp.float32)
    i = jnp.arange(D // 2, dtype=jnp.float32)
    denom = jnp.power(jnp.float32(PE_LAM), 2.0 * i / float(D))
    ang = pos[:, None] / denom[None, :]
    embs = jnp.concatenate([jnp.sin(ang), jnp.cos(ang)], axis=-1)
    ms = jnp.mean(jnp.square(embs), axis=-1, keepdims=True)
    c = embs * jax.lax.rsqrt(ms + EPS)
    return (float(D) ** -0.25) * c


def _vq_body(v_ref, cb_ref, cn_ref, mask_ref, hat_ref, z_ref, err_ref, lc_ref):
    h = pl.program_id(0)
    v = v_ref[0]
    cb = cb_ref[...]

    scores = jax.lax.dot_general(
        v, cb, (((1,), (1,)), ((), ())),
        preferred_element_type=jnp.float32,
        precision=jax.lax.Precision.DEFAULT)
    vnorm2 = jnp.sum(v * v, axis=1)
    d = vnorm2[:, None] - 2.0 * scores + cn_ref[...]
    dmin = jnp.min(d, axis=1, keepdims=True)
    iota = jax.lax.broadcasted_iota(jnp.int32, (L, S), 1)
    z = jnp.min(jnp.where(d == dmin, iota, S), axis=1).astype(jnp.int32)

    errs2 = jnp.maximum(dmin[:, 0], 0.0)

    oh = (iota == z[:, None]).astype(jnp.bfloat16)
    hat = jax.lax.dot_general(
        oh, cb.astype(jnp.bfloat16), (((1,), (0,)), ((), ())),
        preferred_element_type=jnp.float32)

    hat_ref[0] = hat
    z_ref[0, 0] = z
    err_ref[0, 0] = errs2

    part = jnp.sum(mask_ref[...] * errs2[None, :]) * (1.0 / float(B * L))

    @pl.when(h == 0)
    def _():
        lc_ref[0, 0] = 0.0

    lc_ref[0, 0] += part


def kernel(vecs, loss_mask):
    cb = _codebook()
    cn2 = jnp.sum(cb * cb, axis=1)[None, :]
    v = vecs.reshape(H, L, D)
    mask = loss_mask.reshape(1, L)

    hat, z3, err3, lc = pl.pallas_call(
        _vq_body,
        grid=(H,),
        in_specs=[
            pl.BlockSpec((1, L, D), lambda h: (h, 0, 0)),
            pl.BlockSpec((S, D), lambda h: (0, 0)),
            pl.BlockSpec((1, S), lambda h: (0, 0)),
            pl.BlockSpec((1, L), lambda h: (0, 0)),
        ],
        out_specs=[
            pl.BlockSpec((1, L, D), lambda h: (h, 0, 0)),
            pl.BlockSpec((1, 1, L), lambda h: (h, 0, 0)),
            pl.BlockSpec((1, 1, L), lambda h: (h, 0, 0)),
            pl.BlockSpec(memory_space=pltpu.SMEM),
        ],
        out_shape=[
            jax.ShapeDtypeStruct((H, L, D), jnp.float32),
            jax.ShapeDtypeStruct((H, 1, L), jnp.int32),
            jax.ShapeDtypeStruct((H, 1, L), jnp.float32),
            jax.ShapeDtypeStruct((1, 1), jnp.float32),
        ],
    )(v, cb, cn2, mask)

    vecs_hat = hat.reshape(B, H, L, D)
    z = z3.reshape(B, H, L)
    errs2 = err3.reshape(B, H, L)
    l_commit = lc[0, 0]
    l_codebook = jnp.zeros((), dtype=jnp.float32)
    return vecs_hat, z, l_commit, l_codebook, errs2
